# baseline (speedup 1.0000x reference)
import jax
import jax.numpy as jnp
from jax.experimental import pallas as pl
from jax.experimental.pallas import tpu as pltpu

BLOCK_S = 1024


def _add_kernel(x_ref, pos_ref, out_ref):
    pos = pos_ref[...]
    out_ref[:, 0, :] = x_ref[:, 0, :] + pos
    out_ref[:, 1, :] = x_ref[:, 1, :] + pos


def kernel(x, pos_embedding):
    seq_len, batch, d_model = x.shape
    grid = (seq_len // BLOCK_S,)
    return pl.pallas_call(
        _add_kernel,
        grid=grid,
        in_specs=[
            pl.BlockSpec((BLOCK_S, batch, d_model), lambda i: (i, 0, 0)),
            pl.BlockSpec((BLOCK_S, d_model), lambda i: (i, 0)),
        ],
        out_specs=pl.BlockSpec((BLOCK_S, batch, d_model), lambda i: (i, 0, 0)),
        out_shape=jax.ShapeDtypeStruct((seq_len, batch, d_model), x.dtype),
        compiler_params=pltpu.CompilerParams(
            dimension_semantics=("arbitrary",),
        ),
    )(x, pos_embedding)
